# baseline (speedup 1.0000x reference)
import functools

import jax
import jax.numpy as jnp
from jax import lax
from jax.experimental import pallas as pl
from jax.experimental.pallas import tpu as pltpu
from jax.experimental.pallas import tpu_sc as plsc

_info = plsc.get_sparse_core_info()
NC = _info.num_cores
NS = _info.num_subcores
NW = NC * NS
B = 56
DEPTH = 2


def _make_sc_agg(nacc, nb, d, dr0):
    rpt = nacc // NS

    mesh = plsc.VectorSubcoreMesh(core_axis_name="c", subcore_axis_name="s")

    @functools.partial(
        pl.kernel,
        out_type=jax.ShapeDtypeStruct((NC, nacc, d), jnp.float32),
        mesh=mesh,
        scratch_types=[
            pltpu.VMEM((nb, B), jnp.int32),
            pltpu.VMEM((nb, B), jnp.int32),
            *[pltpu.VMEM((B, d), jnp.float32) for _ in range(DEPTH)],
            pltpu.VMEM((80, 128), jnp.float32),
            pltpu.VMEM((80,), jnp.int32),
            pltpu.VMEM_SHARED((nacc, d), jnp.float32),
            *[pltpu.SemaphoreType.DMA for _ in range(2 * DEPTH)],
        ],
        compiler_params=pltpu.CompilerParams(use_tc_tiling_on_sc=False,
                                             needs_layout_passes=False),
    )
    def sc_agg(x_hbm, srcp_hbm, dstp_hbm, zeros_hbm, degidx_hbm, out_hbm,
               src_v, dst_v, *rest):
        rows = list(rest[:DEPTH])
        deg_v, degidx_v, acc_sh = rest[DEPTH:DEPTH + 3]
        gsems = list(rest[DEPTH + 3:DEPTH + 3 + DEPTH])
        ssems = list(rest[DEPTH + 3 + DEPTH:])
        c = lax.axis_index("c")
        s = lax.axis_index("s")
        wid = s * NC + c
        ones16 = jnp.ones((16,), jnp.float32)

        pltpu.sync_copy(zeros_hbm.at[pl.ds(s * rpt, rpt)],
                        acc_sh.at[pl.ds(s * rpt, rpt)])
        pltpu.sync_copy(zeros_hbm.at[pl.ds(0, 80)], deg_v)
        pltpu.sync_copy(degidx_hbm, degidx_v)
        pltpu.sync_copy(srcp_hbm.at[wid], src_v)
        pltpu.sync_copy(dstp_hbm.at[wid], dst_v)
        plsc.subcore_barrier()

        def deg_update(i):
            for g in range(B // 16):
                idx16 = dst_v[i, pl.ds(g * 16, 16)]
                plsc.addupdate_scatter(
                    deg_v,
                    [lax.shift_right_logical(idx16, 7),
                     lax.bitwise_and(idx16, 127)],
                    ones16)

        for p in range(DEPTH - 1):
            pltpu.async_copy(x_hbm.at[src_v.at[p]], rows[p], gsems[p])

        def body(j, _):
            for q in range(DEPTH):
                i = DEPTH * j + q
                qp = (q - 1) % DEPTH
                pltpu.make_async_copy(x_hbm.at[src_v.at[i]], rows[q],
                                      gsems[q]).wait()
                pltpu.async_copy(rows[q], acc_sh.at[dst_v.at[i]], ssems[q],
                                 add=True)
                deg_update(i)

                @pl.when(i > 0)
                def _():
                    pltpu.make_async_copy(rows[qp],
                                          acc_sh.at[dst_v.at[i - 1]],
                                          ssems[qp]).wait()

                @pl.when(i + DEPTH - 1 < nb)
                def _():
                    pltpu.async_copy(x_hbm.at[src_v.at[i + DEPTH - 1]],
                                     rows[qp], gsems[qp])
            return 0

        lax.fori_loop(0, nb // DEPTH, body, 0)
        pltpu.make_async_copy(rows[DEPTH - 1], acc_sh.at[dst_v.at[nb - 1]],
                              ssems[DEPTH - 1]).wait()

        pltpu.sync_copy(deg_v, acc_sh.at[degidx_v], add=True)

        plsc.subcore_barrier()
        pltpu.sync_copy(acc_sh.at[pl.ds(s * rpt, rpt)],
                        out_hbm.at[c, pl.ds(s * rpt, rpt)])

    return sc_agg


def _tc_finish(parts, degp, x, wt, b2, bn):
    n, d = x.shape
    d_out = wt.shape[1]
    nacc = parts.shape[1]

    def body(p_ref, dg_ref, x_ref, wt_ref, b_ref, o_ref):
        acc = p_ref[0] + p_ref[1]
        deg = jnp.maximum(dg_ref[0] + dg_ref[1], 1.0)
        agg = acc / deg
        h = jnp.concatenate([agg, x_ref[...]], axis=1)
        o_ref[...] = (
            jnp.dot(h, wt_ref[...], preferred_element_type=jnp.float32)
            + b_ref[...]
        )

    return pl.pallas_call(
        body,
        grid=(n // bn,),
        in_specs=[
            pl.BlockSpec((NC, bn, d), lambda i: (0, i, 0)),
            pl.BlockSpec((NC, bn, 1), lambda i: (0, i, 0)),
            pl.BlockSpec((bn, d), lambda i: (i, 0)),
            pl.BlockSpec((2 * d, d_out), lambda i: (0, 0)),
            pl.BlockSpec((1, d_out), lambda i: (0, 0)),
        ],
        out_specs=pl.BlockSpec((bn, d_out), lambda i: (i, 0)),
        out_shape=jax.ShapeDtypeStruct((n, d_out), jnp.float32),
    )(parts, degp, x, wt, b2)


def kernel(x, edge_index, num_nodes, W, b):
    n, d = x.shape
    e = edge_index.shape[1]
    ew = -(-e // NW)
    nb = -(-ew // B)
    nb = -(-nb // DEPTH) * DEPTH
    e_pad = NW * nb * B

    dr0 = -(-(n + 1) // 128) * 128
    nacc = -(-(dr0 + 80) // (NS * 8)) * (NS * 8)

    src = edge_index[0]
    dst = edge_index[1]
    if e_pad != e:
        src = jnp.concatenate([src, jnp.zeros((e_pad - e,), src.dtype)])
        dst = jnp.concatenate([dst, jnp.full((e_pad - e,), n, dst.dtype)])
    srcp = src.reshape(NW, nb, B)
    dstp = dst.reshape(NW, nb, B)

    zeros = jnp.zeros((nacc, d), jnp.float32)
    degidx = jnp.arange(dr0, dr0 + 80, dtype=jnp.int32)

    parts = _make_sc_agg(nacc, nb, d, dr0)(x, srcp, dstp, zeros, degidx)

    degp = parts[:, dr0:dr0 + 80, :].reshape(NC, 80 * 128)[:, :n]
    degp = degp.reshape(NC, n, 1)

    wt = W.T
    b2 = b.reshape(1, -1)
    return _tc_finish(parts, degp, x, wt, b2, bn=1000)


# speedup vs baseline: 1.2258x; 1.2258x over previous
import functools

import jax
import jax.numpy as jnp
from jax import lax
from jax.experimental import pallas as pl
from jax.experimental.pallas import tpu as pltpu
from jax.experimental.pallas import tpu_sc as plsc

_info = plsc.get_sparse_core_info()
NC = _info.num_cores
NS = _info.num_subcores
NW = NC * NS
B = 48
DEPTH = 3


def _make_sc_agg(nacc, nb, d, dr0):
    rpt = nacc // NS

    mesh = plsc.VectorSubcoreMesh(core_axis_name="c", subcore_axis_name="s")

    @functools.partial(
        pl.kernel,
        out_type=jax.ShapeDtypeStruct((NC, nacc, d), jnp.float32),
        mesh=mesh,
        scratch_types=[
            pltpu.VMEM((nb, B), jnp.int32),
            pltpu.VMEM((nb, B), jnp.int32),
            *[pltpu.VMEM((B, d), jnp.float32) for _ in range(DEPTH)],
            pltpu.VMEM((80, 128), jnp.float32),
            pltpu.VMEM((80,), jnp.int32),
            pltpu.VMEM_SHARED((nacc, d), jnp.float32),
            *[pltpu.SemaphoreType.DMA for _ in range(2 * DEPTH)],
        ],
        compiler_params=pltpu.CompilerParams(use_tc_tiling_on_sc=False,
                                             needs_layout_passes=False),
    )
    def sc_agg(x_hbm, srcp_hbm, dstp_hbm, zeros_hbm, degidx_hbm, out_hbm,
               src_v, dst_v, *rest):
        rows = list(rest[:DEPTH])
        deg_v, degidx_v, acc_sh = rest[DEPTH:DEPTH + 3]
        gsems = list(rest[DEPTH + 3:DEPTH + 3 + DEPTH])
        ssems = list(rest[DEPTH + 3 + DEPTH:])
        c = lax.axis_index("c")
        s = lax.axis_index("s")
        wid = s * NC + c
        ones16 = jnp.ones((16,), jnp.float32)

        pltpu.sync_copy(zeros_hbm.at[pl.ds(s * rpt, rpt)],
                        acc_sh.at[pl.ds(s * rpt, rpt)])
        pltpu.sync_copy(zeros_hbm.at[pl.ds(0, 80)], deg_v)
        pltpu.sync_copy(degidx_hbm, degidx_v)
        pltpu.sync_copy(srcp_hbm.at[wid], src_v)
        pltpu.sync_copy(dstp_hbm.at[wid], dst_v)
        plsc.subcore_barrier()

        def deg_update(i):
            for g in range(B // 16):
                idx16 = dst_v[i, pl.ds(g * 16, 16)]
                plsc.addupdate_scatter(
                    deg_v,
                    [lax.shift_right_logical(idx16, 7),
                     lax.bitwise_and(idx16, 127)],
                    ones16)

        for p in range(DEPTH - 1):
            pltpu.async_copy(x_hbm.at[src_v.at[p]], rows[p], gsems[p])

        def body(j, _):
            for q in range(DEPTH):
                i = DEPTH * j + q
                qp = (q - 1) % DEPTH
                pltpu.make_async_copy(x_hbm.at[src_v.at[i]], rows[q],
                                      gsems[q]).wait()
                pltpu.async_copy(rows[q], acc_sh.at[dst_v.at[i]], ssems[q],
                                 add=True)
                deg_update(i)

                @pl.when(i > 0)
                def _():
                    pltpu.make_async_copy(rows[qp],
                                          acc_sh.at[dst_v.at[i - 1]],
                                          ssems[qp]).wait()

                @pl.when(i + DEPTH - 1 < nb)
                def _():
                    pltpu.async_copy(x_hbm.at[src_v.at[i + DEPTH - 1]],
                                     rows[qp], gsems[qp])
            return 0

        lax.fori_loop(0, nb // DEPTH, body, 0)
        pltpu.make_async_copy(rows[DEPTH - 1], acc_sh.at[dst_v.at[nb - 1]],
                              ssems[DEPTH - 1]).wait()

        pltpu.sync_copy(deg_v, acc_sh.at[degidx_v], add=True)

        plsc.subcore_barrier()
        pltpu.sync_copy(acc_sh.at[pl.ds(s * rpt, rpt)],
                        out_hbm.at[c, pl.ds(s * rpt, rpt)])

    return sc_agg


def _tc_finish(parts, degp, x, wt, b2, bn):
    n, d = x.shape
    d_out = wt.shape[1]
    nacc = parts.shape[1]

    def body(p_ref, dg_ref, x_ref, wt_ref, b_ref, o_ref):
        acc = p_ref[0] + p_ref[1]
        deg = jnp.maximum(dg_ref[0] + dg_ref[1], 1.0)
        agg = acc / deg
        h = jnp.concatenate([agg, x_ref[...]], axis=1)
        o_ref[...] = (
            jnp.dot(h, wt_ref[...], preferred_element_type=jnp.float32)
            + b_ref[...]
        )

    return pl.pallas_call(
        body,
        grid=(n // bn,),
        in_specs=[
            pl.BlockSpec((NC, bn, d), lambda i: (0, i, 0)),
            pl.BlockSpec((NC, bn, 1), lambda i: (0, i, 0)),
            pl.BlockSpec((bn, d), lambda i: (i, 0)),
            pl.BlockSpec((2 * d, d_out), lambda i: (0, 0)),
            pl.BlockSpec((1, d_out), lambda i: (0, 0)),
        ],
        out_specs=pl.BlockSpec((bn, d_out), lambda i: (i, 0)),
        out_shape=jax.ShapeDtypeStruct((n, d_out), jnp.float32),
    )(parts, degp, x, wt, b2)


def kernel(x, edge_index, num_nodes, W, b):
    n, d = x.shape
    e = edge_index.shape[1]
    ew = -(-e // NW)
    nb = -(-ew // B)
    nb = -(-nb // DEPTH) * DEPTH
    e_pad = NW * nb * B

    dr0 = -(-(n + 1) // 128) * 128
    nacc = -(-(dr0 + 80) // (NS * 8)) * (NS * 8)

    src = edge_index[0]
    dst = edge_index[1]
    if e_pad != e:
        src = jnp.concatenate([src, jnp.zeros((e_pad - e,), src.dtype)])
        dst = jnp.concatenate([dst, jnp.full((e_pad - e,), n, dst.dtype)])
    srcp = src.reshape(NW, nb, B)
    dstp = dst.reshape(NW, nb, B)

    zeros = jnp.zeros((nacc, d), jnp.float32)
    degidx = jnp.arange(dr0, dr0 + 80, dtype=jnp.int32)

    parts = _make_sc_agg(nacc, nb, d, dr0)(x, srcp, dstp, zeros, degidx)

    degp = parts[:, dr0:dr0 + 80, :].reshape(NC, 80 * 128)[:, :n]
    degp = degp.reshape(NC, n, 1)

    wt = W.T
    b2 = b.reshape(1, -1)
    return _tc_finish(parts, degp, x, wt, b2, bn=1000)


# speedup vs baseline: 1.6266x; 1.3270x over previous
import functools

import jax
import jax.numpy as jnp
from jax import lax
from jax.experimental import pallas as pl
from jax.experimental.pallas import tpu as pltpu
from jax.experimental.pallas import tpu_sc as plsc

_info = plsc.get_sparse_core_info()
NC = _info.num_cores
NS = _info.num_subcores
NW = NC * NS
B = 32
DEPTH = 4


def _make_sc_agg(nacc, nb, d, dr0):
    rpt = nacc // NS

    mesh = plsc.VectorSubcoreMesh(core_axis_name="c", subcore_axis_name="s")

    @functools.partial(
        pl.kernel,
        out_type=jax.ShapeDtypeStruct((NC, nacc, d), jnp.float32),
        mesh=mesh,
        scratch_types=[
            pltpu.VMEM((nb, B), jnp.int32),
            pltpu.VMEM((nb, B), jnp.int32),
            *[pltpu.VMEM((B, d), jnp.bfloat16) for _ in range(DEPTH)],
            *[pltpu.VMEM((B, d), jnp.float32) for _ in range(2)],
            pltpu.VMEM((80, 128), jnp.float32),
            pltpu.VMEM((80,), jnp.int32),
            pltpu.VMEM_SHARED((nacc, d), jnp.float32),
            *[pltpu.SemaphoreType.DMA for _ in range(DEPTH + 2)],
        ],
        compiler_params=pltpu.CompilerParams(use_tc_tiling_on_sc=False,
                                             needs_layout_passes=False),
    )
    def sc_agg(x_hbm, srcp_hbm, dstp_hbm, zeros_hbm, degidx_hbm, out_hbm,
               src_v, dst_v, *rest):
        rows = list(rest[:DEPTH])
        fbufs = list(rest[DEPTH:DEPTH + 2])
        deg_v, degidx_v, acc_sh = rest[DEPTH + 2:DEPTH + 5]
        gsems = list(rest[DEPTH + 5:DEPTH + 5 + DEPTH])
        ssems = list(rest[DEPTH + 5 + DEPTH:])
        c = lax.axis_index("c")
        s = lax.axis_index("s")
        wid = s * NC + c
        ones16 = jnp.ones((16,), jnp.float32)

        pltpu.sync_copy(zeros_hbm.at[pl.ds(s * rpt, rpt)],
                        acc_sh.at[pl.ds(s * rpt, rpt)])
        pltpu.sync_copy(zeros_hbm.at[pl.ds(0, 80)], deg_v)
        pltpu.sync_copy(degidx_hbm, degidx_v)
        pltpu.sync_copy(srcp_hbm.at[wid], src_v)
        pltpu.sync_copy(dstp_hbm.at[wid], dst_v)
        plsc.subcore_barrier()

        def deg_update(i):
            for g in range(B // 16):
                idx16 = dst_v[i, pl.ds(g * 16, 16)]
                plsc.addupdate_scatter(
                    deg_v,
                    [lax.shift_right_logical(idx16, 7),
                     lax.bitwise_and(idx16, 127)],
                    ones16)

        def convert(q, f):
            for r in range(B):
                for g in range(d // 32):
                    w = rows[q][r, pl.ds(32 * g, 32)]
                    lo, hi = plsc.unpack(
                        w, format=plsc.PackFormat.INTERLEAVED)
                    fbufs[f][r, pl.ds(32 * g, 16)] = lo
                    fbufs[f][r, pl.ds(32 * g + 16, 16)] = hi

        for p in range(DEPTH):
            pltpu.async_copy(x_hbm.at[src_v.at[p]], rows[p], gsems[p])

        def body(j, _):
            for q in range(DEPTH):
                i = DEPTH * j + q
                f = q % 2
                pltpu.make_async_copy(x_hbm.at[src_v.at[i]], rows[q],
                                      gsems[q]).wait()

                @pl.when(i >= 2)
                def _():
                    pltpu.make_async_copy(fbufs[f],
                                          acc_sh.at[dst_v.at[i - 2]],
                                          ssems[f]).wait()

                convert(q, f)
                pltpu.async_copy(fbufs[f], acc_sh.at[dst_v.at[i]], ssems[f],
                                 add=True)

                @pl.when(i + DEPTH < nb)
                def _():
                    pltpu.async_copy(x_hbm.at[src_v.at[i + DEPTH]],
                                     rows[q], gsems[q])

                deg_update(i)
            return 0

        lax.fori_loop(0, nb // DEPTH, body, 0)
        pltpu.make_async_copy(fbufs[0], acc_sh.at[dst_v.at[nb - 2]],
                              ssems[0]).wait()
        pltpu.make_async_copy(fbufs[1], acc_sh.at[dst_v.at[nb - 1]],
                              ssems[1]).wait()

        pltpu.sync_copy(deg_v, acc_sh.at[degidx_v], add=True)

        plsc.subcore_barrier()
        pltpu.sync_copy(acc_sh.at[pl.ds(s * rpt, rpt)],
                        out_hbm.at[c, pl.ds(s * rpt, rpt)])

    return sc_agg


def _tc_finish(parts, degp, x, wt, b2, bn):
    n, d = x.shape
    d_out = wt.shape[1]
    nacc = parts.shape[1]

    def body(p_ref, dg_ref, x_ref, wt_ref, b_ref, o_ref):
        acc = p_ref[0] + p_ref[1]
        deg = jnp.maximum(dg_ref[0] + dg_ref[1], 1.0)
        agg = acc / deg
        h = jnp.concatenate([agg, x_ref[...]], axis=1)
        o_ref[...] = (
            jnp.dot(h, wt_ref[...], preferred_element_type=jnp.float32)
            + b_ref[...]
        )

    return pl.pallas_call(
        body,
        grid=(n // bn,),
        in_specs=[
            pl.BlockSpec((NC, bn, d), lambda i: (0, i, 0)),
            pl.BlockSpec((NC, bn, 1), lambda i: (0, i, 0)),
            pl.BlockSpec((bn, d), lambda i: (i, 0)),
            pl.BlockSpec((2 * d, d_out), lambda i: (0, 0)),
            pl.BlockSpec((1, d_out), lambda i: (0, 0)),
        ],
        out_specs=pl.BlockSpec((bn, d_out), lambda i: (i, 0)),
        out_shape=jax.ShapeDtypeStruct((n, d_out), jnp.float32),
    )(parts, degp, x, wt, b2)


def kernel(x, edge_index, num_nodes, W, b):
    n, d = x.shape
    e = edge_index.shape[1]
    ew = -(-e // NW)
    nb = -(-ew // B)
    nb = -(-nb // DEPTH) * DEPTH
    e_pad = NW * nb * B

    dr0 = -(-(n + 1) // 128) * 128
    nacc = -(-(dr0 + 80) // (NS * 8)) * (NS * 8)

    src = edge_index[0]
    dst = edge_index[1]
    if e_pad != e:
        src = jnp.concatenate([src, jnp.zeros((e_pad - e,), src.dtype)])
        dst = jnp.concatenate([dst, jnp.full((e_pad - e,), n, dst.dtype)])
    srcp = src.reshape(NW, nb, B)
    dstp = dst.reshape(NW, nb, B)

    zeros = jnp.zeros((nacc, d), jnp.float32)
    degidx = jnp.arange(dr0, dr0 + 80, dtype=jnp.int32)

    xh = x.astype(jnp.bfloat16)
    parts = _make_sc_agg(nacc, nb, d, dr0)(xh, srcp, dstp, zeros, degidx)

    degp = parts[:, dr0:dr0 + 80, :].reshape(NC, 80 * 128)[:, :n]
    degp = degp.reshape(NC, n, 1)

    perm = (jnp.arange(d // 32).repeat(32) * 32
            + jnp.tile(jnp.concatenate([jnp.arange(16) * 2,
                                        jnp.arange(16) * 2 + 1]), d // 32))
    wt = W.T
    wt = jnp.concatenate([wt[:d][perm], wt[d:]], axis=0)
    b2 = b.reshape(1, -1)
    return _tc_finish(parts, degp, x, wt, b2, bn=1000)


# speedup vs baseline: 1.6785x; 1.0319x over previous
import functools

import jax
import jax.numpy as jnp
from jax import lax
from jax.experimental import pallas as pl
from jax.experimental.pallas import tpu as pltpu
from jax.experimental.pallas import tpu_sc as plsc

_info = plsc.get_sparse_core_info()
NC = _info.num_cores
NS = _info.num_subcores
NW = NC * NS
B = 48
DEPTH = 2


def _make_sc_agg(nacc, nb, d, dr0):
    rpt = nacc // NS

    mesh = plsc.VectorSubcoreMesh(core_axis_name="c", subcore_axis_name="s")

    @functools.partial(
        pl.kernel,
        out_type=jax.ShapeDtypeStruct((NC, nacc, d), jnp.float32),
        mesh=mesh,
        scratch_types=[
            pltpu.VMEM((nb, B), jnp.int32),
            pltpu.VMEM((nb, B), jnp.int32),
            *[pltpu.VMEM((B, d), jnp.bfloat16) for _ in range(DEPTH)],
            *[pltpu.VMEM((B, d), jnp.float32) for _ in range(2)],
            pltpu.VMEM((80, 128), jnp.float32),
            pltpu.VMEM((80,), jnp.int32),
            pltpu.VMEM_SHARED((nacc, d), jnp.float32),
            *[pltpu.SemaphoreType.DMA for _ in range(DEPTH + 2)],
        ],
        compiler_params=pltpu.CompilerParams(use_tc_tiling_on_sc=False,
                                             needs_layout_passes=False),
    )
    def sc_agg(x_hbm, srcp_hbm, dstp_hbm, zeros_hbm, degidx_hbm, out_hbm,
               src_v, dst_v, *rest):
        rows = list(rest[:DEPTH])
        fbufs = list(rest[DEPTH:DEPTH + 2])
        deg_v, degidx_v, acc_sh = rest[DEPTH + 2:DEPTH + 5]
        gsems = list(rest[DEPTH + 5:DEPTH + 5 + DEPTH])
        ssems = list(rest[DEPTH + 5 + DEPTH:])
        c = lax.axis_index("c")
        s = lax.axis_index("s")
        wid = s * NC + c
        ones16 = jnp.ones((16,), jnp.float32)

        pltpu.sync_copy(zeros_hbm.at[pl.ds(s * rpt, rpt)],
                        acc_sh.at[pl.ds(s * rpt, rpt)])
        pltpu.sync_copy(zeros_hbm.at[pl.ds(0, 80)], deg_v)
        pltpu.sync_copy(degidx_hbm, degidx_v)
        pltpu.sync_copy(srcp_hbm.at[wid], src_v)
        pltpu.sync_copy(dstp_hbm.at[wid], dst_v)
        plsc.subcore_barrier()

        def deg_update(i):
            for g in range(B // 16):
                idx16 = dst_v[i, pl.ds(g * 16, 16)]
                plsc.addupdate_scatter(
                    deg_v,
                    [lax.shift_right_logical(idx16, 7),
                     lax.bitwise_and(idx16, 127)],
                    ones16)

        def convert(q, f):
            for r in range(B):
                for g in range(d // 32):
                    w = rows[q][r, pl.ds(32 * g, 32)]
                    lo, hi = plsc.unpack(
                        w, format=plsc.PackFormat.INTERLEAVED)
                    fbufs[f][r, pl.ds(32 * g, 16)] = lo
                    fbufs[f][r, pl.ds(32 * g + 16, 16)] = hi

        for p in range(DEPTH):
            pltpu.async_copy(x_hbm.at[src_v.at[p]], rows[p], gsems[p])

        def body(j, _):
            for q in range(DEPTH):
                i = DEPTH * j + q
                f = q % 2
                pltpu.make_async_copy(x_hbm.at[src_v.at[i]], rows[q],
                                      gsems[q]).wait()

                @pl.when(i >= 2)
                def _():
                    pltpu.make_async_copy(fbufs[f],
                                          acc_sh.at[dst_v.at[i - 2]],
                                          ssems[f]).wait()

                convert(q, f)
                pltpu.async_copy(fbufs[f], acc_sh.at[dst_v.at[i]], ssems[f],
                                 add=True)

                @pl.when(i + DEPTH < nb)
                def _():
                    pltpu.async_copy(x_hbm.at[src_v.at[i + DEPTH]],
                                     rows[q], gsems[q])

                deg_update(i)
            return 0

        lax.fori_loop(0, nb // DEPTH, body, 0)
        pltpu.make_async_copy(fbufs[0], acc_sh.at[dst_v.at[nb - 2]],
                              ssems[0]).wait()
        pltpu.make_async_copy(fbufs[1], acc_sh.at[dst_v.at[nb - 1]],
                              ssems[1]).wait()

        pltpu.sync_copy(deg_v, acc_sh.at[degidx_v], add=True)

        plsc.subcore_barrier()
        pltpu.sync_copy(acc_sh.at[pl.ds(s * rpt, rpt)],
                        out_hbm.at[c, pl.ds(s * rpt, rpt)])

    return sc_agg


def _tc_finish(parts, degp, x, wt, b2, bn):
    n, d = x.shape
    d_out = wt.shape[1]
    nacc = parts.shape[1]

    def body(p_ref, dg_ref, x_ref, wt_ref, b_ref, o_ref):
        acc = p_ref[0] + p_ref[1]
        deg = jnp.maximum(dg_ref[0] + dg_ref[1], 1.0)
        agg = acc / deg
        h = jnp.concatenate([agg, x_ref[...]], axis=1)
        o_ref[...] = (
            jnp.dot(h, wt_ref[...], preferred_element_type=jnp.float32)
            + b_ref[...]
        )

    return pl.pallas_call(
        body,
        grid=(n // bn,),
        in_specs=[
            pl.BlockSpec((NC, bn, d), lambda i: (0, i, 0)),
            pl.BlockSpec((NC, bn, 1), lambda i: (0, i, 0)),
            pl.BlockSpec((bn, d), lambda i: (i, 0)),
            pl.BlockSpec((2 * d, d_out), lambda i: (0, 0)),
            pl.BlockSpec((1, d_out), lambda i: (0, 0)),
        ],
        out_specs=pl.BlockSpec((bn, d_out), lambda i: (i, 0)),
        out_shape=jax.ShapeDtypeStruct((n, d_out), jnp.float32),
    )(parts, degp, x, wt, b2)


def kernel(x, edge_index, num_nodes, W, b):
    n, d = x.shape
    e = edge_index.shape[1]
    ew = -(-e // NW)
    nb = -(-ew // B)
    nb = -(-nb // DEPTH) * DEPTH
    e_pad = NW * nb * B

    dr0 = -(-(n + 1) // 128) * 128
    nacc = -(-(dr0 + 80) // (NS * 8)) * (NS * 8)

    src = edge_index[0]
    dst = edge_index[1]
    if e_pad != e:
        src = jnp.concatenate([src, jnp.zeros((e_pad - e,), src.dtype)])
        dst = jnp.concatenate([dst, jnp.full((e_pad - e,), n, dst.dtype)])
    srcp = src.reshape(NW, nb, B)
    dstp = dst.reshape(NW, nb, B)

    zeros = jnp.zeros((nacc, d), jnp.float32)
    degidx = jnp.arange(dr0, dr0 + 80, dtype=jnp.int32)

    xh = x.astype(jnp.bfloat16)
    parts = _make_sc_agg(nacc, nb, d, dr0)(xh, srcp, dstp, zeros, degidx)

    degp = parts[:, dr0:dr0 + 80, :].reshape(NC, 80 * 128)[:, :n]
    degp = degp.reshape(NC, n, 1)

    perm = (jnp.arange(d // 32).repeat(32) * 32
            + jnp.tile(jnp.concatenate([jnp.arange(16) * 2,
                                        jnp.arange(16) * 2 + 1]), d // 32))
    wt = W.T
    wt = jnp.concatenate([wt[:d][perm], wt[d:]], axis=0)
    b2 = b.reshape(1, -1)
    return _tc_finish(parts, degp, x, wt, b2, bn=1000)


# speedup vs baseline: 1.6969x; 1.0110x over previous
import functools

import jax
import jax.numpy as jnp
from jax import lax
from jax.experimental import pallas as pl
from jax.experimental.pallas import tpu as pltpu
from jax.experimental.pallas import tpu_sc as plsc

_info = plsc.get_sparse_core_info()
NC = _info.num_cores
NS = _info.num_subcores
NW = NC * NS
B = 48
DEPTH = 2


def _make_sc_agg(nacc, nb, d, dr0):
    rpt = nacc // NS

    mesh = plsc.VectorSubcoreMesh(core_axis_name="c", subcore_axis_name="s")

    @functools.partial(
        pl.kernel,
        out_type=jax.ShapeDtypeStruct((NC, nacc, d), jnp.float32),
        mesh=mesh,
        scratch_types=[
            pltpu.VMEM((nb, B), jnp.int32),
            pltpu.VMEM((nb, B), jnp.int32),
            *[pltpu.VMEM((B, d), jnp.bfloat16) for _ in range(DEPTH)],
            *[pltpu.VMEM((B, d), jnp.float32) for _ in range(2)],
            pltpu.VMEM((80, 128), jnp.float32),
            pltpu.VMEM((80,), jnp.int32),
            pltpu.VMEM_SHARED((nacc, d), jnp.float32),
            *[pltpu.SemaphoreType.DMA for _ in range(DEPTH + 2)],
        ],
        compiler_params=pltpu.CompilerParams(use_tc_tiling_on_sc=False,
                                             needs_layout_passes=False),
    )
    def sc_agg(x_hbm, srcp_hbm, dstp_hbm, zeros_hbm, degidx_hbm, out_hbm,
               src_v, dst_v, *rest):
        rows = list(rest[:DEPTH])
        fbufs = list(rest[DEPTH:DEPTH + 2])
        deg_v, degidx_v, acc_sh = rest[DEPTH + 2:DEPTH + 5]
        gsems = list(rest[DEPTH + 5:DEPTH + 5 + DEPTH])
        ssems = list(rest[DEPTH + 5 + DEPTH:])
        c = lax.axis_index("c")
        s = lax.axis_index("s")
        wid = s * NC + c
        ones16 = jnp.ones((16,), jnp.float32)

        pltpu.sync_copy(zeros_hbm.at[pl.ds(s * rpt, rpt)],
                        acc_sh.at[pl.ds(s * rpt, rpt)])
        pltpu.sync_copy(zeros_hbm.at[pl.ds(0, 80)], deg_v)
        pltpu.sync_copy(degidx_hbm, degidx_v)
        pltpu.sync_copy(srcp_hbm.at[wid], src_v)
        pltpu.sync_copy(dstp_hbm.at[wid], dst_v)
        plsc.subcore_barrier()

        def deg_update(i):
            for g in range(B // 16):
                idx16 = dst_v[i, pl.ds(g * 16, 16)]
                plsc.addupdate_scatter(
                    deg_v,
                    [lax.shift_right_logical(idx16, 7),
                     lax.bitwise_and(idx16, 127)],
                    ones16)

        def convert(q, f):
            for r in range(B):
                for g in range(d // 32):
                    w = rows[q][r, pl.ds(32 * g, 32)]
                    lo, hi = plsc.unpack(
                        w, format=plsc.PackFormat.INTERLEAVED)
                    fbufs[f][r, pl.ds(32 * g, 16)] = lo
                    fbufs[f][r, pl.ds(32 * g + 16, 16)] = hi

        for p in range(DEPTH):
            pltpu.async_copy(x_hbm.at[src_v.at[p]], rows[p], gsems[p])

        def body(j, _):
            for q in range(DEPTH):
                i = DEPTH * j + q
                f = q % 2
                pltpu.make_async_copy(x_hbm.at[src_v.at[i]], rows[q],
                                      gsems[q]).wait()

                @pl.when(i >= 2)
                def _():
                    pltpu.make_async_copy(fbufs[f],
                                          acc_sh.at[dst_v.at[i - 2]],
                                          ssems[f]).wait()

                convert(q, f)
                pltpu.async_copy(fbufs[f], acc_sh.at[dst_v.at[i]], ssems[f],
                                 add=True)

                @pl.when(i + DEPTH < nb)
                def _():
                    pltpu.async_copy(x_hbm.at[src_v.at[i + DEPTH]],
                                     rows[q], gsems[q])

                deg_update(i)
            return 0

        lax.fori_loop(0, nb // DEPTH, body, 0)
        pltpu.make_async_copy(fbufs[0], acc_sh.at[dst_v.at[nb - 2]],
                              ssems[0]).wait()
        pltpu.make_async_copy(fbufs[1], acc_sh.at[dst_v.at[nb - 1]],
                              ssems[1]).wait()

        pltpu.sync_copy(deg_v, acc_sh.at[degidx_v], add=True)

        plsc.subcore_barrier()
        pltpu.sync_copy(acc_sh.at[pl.ds(s * rpt, rpt)],
                        out_hbm.at[c, pl.ds(s * rpt, rpt)])

    return sc_agg


def _tc_finish(parts, degp, x, wt, b2, bn):
    n, d = x.shape
    d_out = wt.shape[1]
    nacc = parts.shape[1]

    def body(p_ref, dg_ref, x_ref, wt_ref, b_ref, o_ref):
        acc = p_ref[0] + p_ref[1]
        deg = jnp.maximum(dg_ref[0] + dg_ref[1], 1.0)
        agg = acc / deg
        h = jnp.concatenate([agg, x_ref[...]], axis=1)
        o_ref[...] = (
            jnp.dot(h, wt_ref[...], preferred_element_type=jnp.float32)
            + b_ref[...]
        )

    return pl.pallas_call(
        body,
        grid=(n // bn,),
        in_specs=[
            pl.BlockSpec((NC, bn, d), lambda i: (0, i, 0)),
            pl.BlockSpec((NC, bn, 1), lambda i: (0, i, 0)),
            pl.BlockSpec((bn, d), lambda i: (i, 0)),
            pl.BlockSpec((2 * d, d_out), lambda i: (0, 0)),
            pl.BlockSpec((1, d_out), lambda i: (0, 0)),
        ],
        out_specs=pl.BlockSpec((bn, d_out), lambda i: (i, 0)),
        out_shape=jax.ShapeDtypeStruct((n, d_out), jnp.float32),
    )(parts, degp, x, wt, b2)


def kernel(x, edge_index, num_nodes, W, b):
    n, d = x.shape
    e = edge_index.shape[1]
    ew = -(-e // NW)
    nb = -(-ew // B)
    nb = -(-nb // DEPTH) * DEPTH
    e_pad = NW * nb * B

    dr0 = -(-(n + 1) // 128) * 128
    nacc = -(-(dr0 + 80) // (NS * 8)) * (NS * 8)

    src = edge_index[0]
    dst = edge_index[1]
    if e_pad != e:
        src = jnp.concatenate([src, jnp.zeros((e_pad - e,), src.dtype)])
        dst = jnp.concatenate([dst, jnp.full((e_pad - e,), n, dst.dtype)])
    srcp = src.reshape(NW, nb, B)
    dstp = dst.reshape(NW, nb, B)

    zeros = jnp.zeros((nacc, d), jnp.float32)
    degidx = jnp.arange(dr0, dr0 + 80, dtype=jnp.int32)

    xh = x.astype(jnp.bfloat16)
    parts = _make_sc_agg(nacc, nb, d, dr0)(xh, srcp, dstp, zeros, degidx)

    degp = parts[:, dr0:dr0 + 80, :].reshape(NC, 80 * 128)[:, :n]
    degp = degp.reshape(NC, n, 1)

    perm = (jnp.arange(d // 32).repeat(32) * 32
            + jnp.tile(jnp.concatenate([jnp.arange(16) * 2,
                                        jnp.arange(16) * 2 + 1]), d // 32))
    wt = W.T
    wt = jnp.concatenate([wt[:d][perm], wt[d:]], axis=0)
    b2 = b.reshape(1, -1)
    return _tc_finish(parts, degp, x, wt, b2, bn=2000)
